# Initial kernel scaffold; baseline (speedup 1.0000x reference)
import jax
import jax.numpy as jnp
from jax.experimental import pallas as pl


def kernel(pitch, amplitude, attention, wavetables):
    raise NotImplementedError("write your pallas kernel here")


# speedup vs baseline: 8.2454x; 8.2454x over previous
import functools

import jax
import jax.numpy as jnp
from jax import lax
from jax.experimental import pallas as pl
from jax.experimental.pallas import tpu as pltpu
from jax.experimental.pallas import tpu_sc as plsc

SR = 44100
N_WT = 20
WT_LEN = 512
B = 8
T = 64000

CHUNK = 512
NCHUNKS = T // CHUNK
NRPAD = 520

NTILES = 32
PER_TILE = (B * T) // NTILES
BLK = 2000
NBLK = PER_TILE // BLK
GROUPS = BLK // 16


def _tc_index_kernel(pitch_ref, wt_ref, idx_ref, tab_ref, carry_ref):
    c = pl.program_id(0)

    @pl.when(c == 0)
    def _init():
        carry_ref[...] = jnp.zeros((8, 128), jnp.float32)
        wt = wt_ref[...]
        row = lax.broadcasted_iota(jnp.int32, (N_WT, WT_LEN), 0)
        eff = jnp.where(row < 4, wt, jnp.tanh(wt))
        tab_ref[:, :WT_LEN] = eff
        tab_ref[:, WT_LEN:WT_LEN + 1] = eff[:, :1]
        tab_ref[:, WT_LEN + 1:] = jnp.zeros((N_WT, NRPAD - WT_LEN - 1), jnp.float32)

    inc = pitch_ref[...] / jnp.float32(SR) * jnp.float32(WT_LEN)
    lane = lax.broadcasted_iota(jnp.int32, (8, CHUNK), 1)
    inc = jnp.where((c == 0) & (lane == 0), jnp.float32(0.0), inc)
    x = inc
    d = 1
    while d < CHUNK:
        shifted = jnp.concatenate(
            [jnp.zeros((8, d), jnp.float32), x[:, :CHUNK - d]], axis=1)
        x = x + shifted
        d *= 2
    carry = carry_ref[:, :1]
    y = carry + x
    idx_ref[...] = y - jnp.float32(512.0) * jnp.floor(y * jnp.float32(1.0 / 512.0))
    ynew = carry + x[:, CHUNK - 1:CHUNK]
    cnew = ynew - jnp.float32(512.0) * jnp.floor(ynew * jnp.float32(1.0 / 512.0))
    carry_ref[...] = jnp.broadcast_to(cnew, (8, 128))


def _tc_index(pitch2d, wavetables):
    return pl.pallas_call(
        _tc_index_kernel,
        grid=(NCHUNKS,),
        in_specs=[
            pl.BlockSpec((B, CHUNK), lambda c: (0, c)),
            pl.BlockSpec((N_WT, WT_LEN), lambda c: (0, 0)),
        ],
        out_specs=[
            pl.BlockSpec((B, CHUNK), lambda c: (0, c)),
            pl.BlockSpec((N_WT, NRPAD), lambda c: (0, 0)),
        ],
        out_shape=[
            jax.ShapeDtypeStruct((B, T), jnp.float32),
            jax.ShapeDtypeStruct((N_WT, NRPAD), jnp.float32),
        ],
        scratch_shapes=[pltpu.VMEM((8, 128), jnp.float32)],
    )(pitch2d, wavetables)


def _sc_body(idx_hbm, amp_hbm, att_hbm, tab_hbm, out_hbm,
             tab_v, idx_v, amp_v, att_v, out_v):
    cid = lax.axis_index("c")
    sid = lax.axis_index("s")
    wid = sid * 2 + cid
    pltpu.sync_copy(tab_hbm, tab_v)
    base = wid * PER_TILE
    lane20 = jnp.arange(16, dtype=jnp.int32) * 20

    def block_body(b, carry):
        off = base + b * BLK
        pltpu.sync_copy(idx_hbm.at[pl.ds(off, BLK)], idx_v)
        pltpu.sync_copy(amp_hbm.at[pl.ds(off, BLK)], amp_v)
        pltpu.sync_copy(att_hbm.at[pl.ds(off * 20, BLK * 20)], att_v)

        def group(g, c2):
            s = g * 16
            idxf = idx_v[pl.ds(s, 16)]
            il = idxf.astype(jnp.int32)
            alpha = idxf - il.astype(jnp.float32)
            ab = lane20 + g * 320
            acc = jnp.zeros((16,), jnp.float32)
            for k in range(N_WT):
                tk = il + k * NRPAD
                lo = plsc.load_gather(tab_v, [tk])
                hi = plsc.load_gather(tab_v, [tk + 1])
                a = plsc.load_gather(att_v, [ab + k])
                acc = acc + a * (lo + alpha * (hi - lo))
            out_v[pl.ds(s, 16)] = acc * amp_v[pl.ds(s, 16)]
            return c2

        lax.fori_loop(0, GROUPS, group, 0)
        pltpu.sync_copy(out_v, out_hbm.at[pl.ds(off, BLK)])
        return carry

    lax.fori_loop(0, NBLK, block_body, 0)


def _sc_gather(idx_flat, amp_flat, att_flat, tab_flat):
    mesh = plsc.VectorSubcoreMesh(
        core_axis_name="c", subcore_axis_name="s", num_cores=2, num_subcores=16)
    fn = pl.kernel(
        _sc_body,
        out_type=jax.ShapeDtypeStruct((B * T,), jnp.float32),
        mesh=mesh,
        compiler_params=pltpu.CompilerParams(needs_layout_passes=False),
        scratch_types=[
            pltpu.VMEM((N_WT * NRPAD,), jnp.float32),
            pltpu.VMEM((BLK,), jnp.float32),
            pltpu.VMEM((BLK,), jnp.float32),
            pltpu.VMEM((BLK * 20,), jnp.float32),
            pltpu.VMEM((BLK,), jnp.float32),
        ],
    )
    return fn(idx_flat, amp_flat, att_flat, tab_flat)


def kernel(pitch, amplitude, attention, wavetables):
    pitch2d = jnp.squeeze(pitch, -1)
    idx_arr, tab = _tc_index(pitch2d, wavetables)
    out_flat = _sc_gather(
        idx_arr.reshape(-1),
        amplitude.reshape(-1),
        attention.reshape(-1),
        tab.reshape(-1),
    )
    return out_flat.reshape(B, T, 1)
